# baseline (speedup 1.0000x reference)
import functools

import jax
import jax.numpy as jnp
from jax.experimental import pallas as pl
from jax.experimental.pallas import tpu as pltpu

K = 25
PAD = (K - 1) // 2
NE = 8
B, L, P = 1024, 1024, 1024
HID = 64
UW = 0.3


def _moe_kernel(z_ref, cov_ref, wear_ref, rw1_ref, rb1_ref, rw2_ref, rb2_ref,
                bs_ref, bt_ref, b1_ref, b2_ref,
                ws_hbm, wt_hbm, w1_hbm, w2_hbm,
                out_ref, res_ref, mm_ref, wvec_ref, dec_ref,
                wsbuf, wtbuf, w1buf, w2buf,
                ws_sem, wt_sem, w1_sem, w2_sem):
    i = pl.program_id(0)
    dn = (((1,), (1,)), ((), ()))
    f32 = jnp.float32
    parity = jax.lax.rem(i, 2)

    def start_dec_copies(e, slot):
        pltpu.make_async_copy(ws_hbm.at[e], wsbuf.at[slot],
                              ws_sem.at[slot]).start()
        pltpu.make_async_copy(wt_hbm.at[e], wtbuf.at[slot],
                              wt_sem.at[slot]).start()

    def wait_dec_copies(e, slot):
        pltpu.make_async_copy(ws_hbm.at[e], wsbuf.at[slot],
                              ws_sem.at[slot]).wait()
        pltpu.make_async_copy(wt_hbm.at[e], wtbuf.at[slot],
                              wt_sem.at[slot]).wait()

    def start_mlp_copies(e, slot):
        pltpu.make_async_copy(w1_hbm.at[e], w1buf.at[slot],
                              w1_sem.at[slot]).start()
        pltpu.make_async_copy(w2_hbm.at[e], w2buf.at[slot],
                              w2_sem.at[slot]).start()

    def wait_mlp_copies(e, slot):
        pltpu.make_async_copy(w1_hbm.at[e], w1buf.at[slot],
                              w1_sem.at[slot]).wait()
        pltpu.make_async_copy(w2_hbm.at[e], w2buf.at[slot],
                              w2_sem.at[slot]).wait()

    @pl.when(i == 0)
    def _prep():
        start_dec_copies(0, 0)
        start_dec_copies(1, 1)
        Z = z_ref[...]
        jc = jax.lax.broadcasted_iota(jnp.int32, (L, L), 0).astype(f32)
        lc = jax.lax.broadcasted_iota(jnp.int32, (L, L), 1).astype(f32)
        band = (jnp.abs(jc - lc) <= PAD).astype(f32)
        front = jnp.where(lc == 0, jnp.maximum(PAD - jc, 0.0), 0.0)
        back = jnp.where(lc == L - 1, jnp.maximum(jc - (L - 1 - PAD), 0.0), 0.0)
        A = (band + front + back) * (1.0 / K)
        mm = jax.lax.dot_general(Z, A, dn, preferred_element_type=f32)
        mm_ref[...] = mm
        res_ref[...] = Z - mm
        comb = jnp.concatenate([cov_ref[...], wear_ref[...]], axis=1)
        hr = jnp.maximum(
            jax.lax.dot_general(comb, rw1_ref[...], dn,
                                preferred_element_type=f32) + rb1_ref[...],
            0.0)
        logits = jax.lax.dot_general(hr, rw2_ref[...], dn,
                                     preferred_element_type=f32) + rb2_ref[...]
        mx = jnp.max(logits, axis=1, keepdims=True)
        e = jnp.exp(logits - mx)
        sm = e / jnp.sum(e, axis=1, keepdims=True) * (1.0 - UW)
        wvec_ref[...] = jnp.concatenate(
            [jnp.full((B, 1), UW, f32), sm], axis=1)
        dec_ref[1] = jnp.zeros((B, P), f32)
        w1buf[1] = jnp.zeros((HID, P), f32)
        w2buf[1] = jnp.zeros((P, HID), f32)
        out_ref[...] = jax.lax.dot_general(
            wvec_ref[...], b2_ref[...], (((1,), (0,)), ((), ())),
            preferred_element_type=f32)

    @pl.when(jnp.logical_and(i > 0, i < NE - 1))
    def _ahead():
        start_dec_copies(i + 1, 1 - parity)

    @pl.when(i < NE)
    def _decode():
        start_mlp_copies(i, parity)
        wait_dec_copies(i, parity)
        dec_ref[parity] = (
            jax.lax.dot_general(res_ref[...], wsbuf[parity], dn,
                                preferred_element_type=f32)
            + jax.lax.dot_general(mm_ref[...], wtbuf[parity], dn,
                                  preferred_element_type=f32)
            + bs_ref[0] + bt_ref[0])

    @pl.when(i > 0)
    def _wait_mlp():
        wait_mlp_copies(i - 1, 1 - parity)

    d = dec_ref[1 - parity]
    h = jnp.maximum(
        jax.lax.dot_general(d, w1buf[1 - parity], dn,
                            preferred_element_type=f32) + b1_ref[0], 0.0)
    lane = jax.lax.broadcasted_iota(jnp.int32, (1, NE), 1)
    w = jnp.sum(wvec_ref[...] * (lane == (i - 1)).astype(f32),
                axis=1, keepdims=True)
    g = w * h
    o = jax.lax.dot_general(g, w2buf[1 - parity], dn,
                            preferred_element_type=f32)
    out_ref[...] += o


@functools.partial(jax.jit, static_argnames=())
def kernel(z, cov_embedding, wearable_embedding, expert_Ws, expert_bs,
           expert_Wt, expert_bt, expert_W1, expert_b1, expert_W2, expert_b2,
           router_W1, router_b1, router_W2, router_b2):
    zsq = z[:, :, 0]
    rb1 = router_b1.reshape(1, HID)
    rb2 = router_b2.reshape(1, NE - 1)
    bsr = expert_bs.reshape(NE, 1, P)
    btr = expert_bt.reshape(NE, 1, P)
    b1r = expert_b1.reshape(NE, 1, HID)

    def dec_idx(i):
        return (jnp.minimum(i, NE - 1), 0, 0)

    def mlp_idx(i):
        return (jnp.maximum(i - 1, 0), 0, 0)

    out = pl.pallas_call(
        _moe_kernel,
        grid=(NE + 1,),
        in_specs=[
            pl.BlockSpec((B, L), lambda i: (0, 0)),
            pl.BlockSpec((B, HID), lambda i: (0, 0)),
            pl.BlockSpec((B, HID), lambda i: (0, 0)),
            pl.BlockSpec((HID, 2 * HID), lambda i: (0, 0)),
            pl.BlockSpec((1, HID), lambda i: (0, 0)),
            pl.BlockSpec((NE - 1, HID), lambda i: (0, 0)),
            pl.BlockSpec((1, NE - 1), lambda i: (0, 0)),
            pl.BlockSpec((1, 1, P), dec_idx),
            pl.BlockSpec((1, 1, P), dec_idx),
            pl.BlockSpec((1, 1, HID), mlp_idx),
            pl.BlockSpec((NE, P), lambda i: (0, 0)),
            pl.BlockSpec(memory_space=pl.ANY),
            pl.BlockSpec(memory_space=pl.ANY),
            pl.BlockSpec(memory_space=pl.ANY),
            pl.BlockSpec(memory_space=pl.ANY),
        ],
        out_specs=pl.BlockSpec((B, P), lambda i: (0, 0)),
        out_shape=jax.ShapeDtypeStruct((B, P), jnp.float32),
        compiler_params=pltpu.CompilerParams(
            dimension_semantics=("arbitrary",),
            vmem_limit_bytes=100 * 1024 * 1024,
        ),
        scratch_shapes=[
            pltpu.VMEM((B, L), jnp.float32),
            pltpu.VMEM((B, L), jnp.float32),
            pltpu.VMEM((B, NE), jnp.float32),
            pltpu.VMEM((2, B, P), jnp.float32),
            pltpu.VMEM((2, P, L), jnp.float32),
            pltpu.VMEM((2, P, L), jnp.float32),
            pltpu.VMEM((2, HID, P), jnp.float32),
            pltpu.VMEM((2, P, HID), jnp.float32),
            pltpu.SemaphoreType.DMA((2,)),
            pltpu.SemaphoreType.DMA((2,)),
            pltpu.SemaphoreType.DMA((2,)),
            pltpu.SemaphoreType.DMA((2,)),
        ],
    )(zsq, cov_embedding, wearable_embedding, router_W1, rb1, router_W2, rb2,
      bsr, btr, b1r, expert_b2,
      expert_Ws, expert_Wt, expert_W1, expert_W2)

    return out[..., None]
